# Initial kernel scaffold; baseline (speedup 1.0000x reference)
import jax
import jax.numpy as jnp
from jax.experimental import pallas as pl


def kernel(x, edge_index, edge_attr, u, batch, W1, b1, W2, b2):
    raise NotImplementedError("write your pallas kernel here")


# speedup vs baseline: 6.5369x; 6.5369x over previous
import jax
import jax.numpy as jnp
from jax import lax
from jax.experimental import pallas as pl
from jax.experimental.pallas import tpu as pltpu
from jax.experimental.pallas import tpu_sc as plsc

N_NODES = 10000
N_EDGES = 320000
D_FEAT = 128
D_EDGE = 16
HIDDEN = 128
D_OUT = 128

NC, NS, L = 2, 16, 16
NW = NC * NS
EPW = N_EDGES // NW
SUB = 125
NSUB = EPW // SUB
STAGE = 2000
NST = EPW // STAGE
SPS = STAGE // SUB
RPT = 632
NPAD = NS * RPT


def _sc_scatter_body(idx_hbm, attr_hbm, psums_hbm, pcnts_hbm,
                     idx_vm, attr_vm, ones_vm, zero_vm, sums_sh, cnts_sh):
    c = lax.axis_index("c")
    s = lax.axis_index("s")
    wid = c * NS + s

    def fill_ones(i, carry):
        ones_vm[i] = jnp.full((L,), 1.0, jnp.float32)
        return carry

    lax.fori_loop(0, SUB, fill_ones, 0)

    def fill_zero(i, carry):
        zero_vm[i] = jnp.zeros((L,), jnp.float32)
        return carry

    lax.fori_loop(0, RPT, fill_zero, 0)

    pltpu.sync_copy(zero_vm, sums_sh.at[pl.ds(s * RPT, RPT)])
    pltpu.sync_copy(zero_vm, cnts_sh.at[pl.ds(s * RPT, RPT)])
    plsc.subcore_barrier()

    pltpu.sync_copy(idx_hbm.at[pl.ds(wid * NSUB, NSUB)], idx_vm)

    for st in range(NST):
        base = wid * EPW + st * STAGE
        pltpu.sync_copy(attr_hbm.at[pl.ds(base, STAGE)], attr_vm)

        def scat(j, carry):
            row = st * SPS + j
            pltpu.sync_copy(attr_vm.at[pl.ds(j * SUB, SUB)],
                            sums_sh.at[idx_vm.at[row]], add=True)
            pltpu.sync_copy(ones_vm, cnts_sh.at[idx_vm.at[row]], add=True)
            return carry

        lax.fori_loop(0, SPS, scat, 0)

    plsc.subcore_barrier()
    pltpu.sync_copy(sums_sh.at[pl.ds(s * RPT, RPT)],
                    psums_hbm.at[c, pl.ds(s * RPT, RPT)])
    pltpu.sync_copy(cnts_sh.at[pl.ds(s * RPT, RPT)],
                    pcnts_hbm.at[c, pl.ds(s * RPT, RPT)])


def _make_scatter():
    mesh = plsc.VectorSubcoreMesh(core_axis_name="c", subcore_axis_name="s",
                                  num_cores=NC, num_subcores=NS)
    return pl.kernel(
        _sc_scatter_body,
        out_type=[jax.ShapeDtypeStruct((NC, NPAD, D_EDGE), jnp.float32),
                  jax.ShapeDtypeStruct((NC, NPAD, D_EDGE), jnp.float32)],
        mesh=mesh,
        scratch_types=[
            pltpu.VMEM((NSUB, SUB), jnp.int32),
            pltpu.VMEM((STAGE, D_EDGE), jnp.float32),
            pltpu.VMEM((SUB, D_EDGE), jnp.float32),
            pltpu.VMEM((RPT, D_EDGE), jnp.float32),
            pltpu.VMEM_SHARED((NPAD, D_EDGE), jnp.float32),
            pltpu.VMEM_SHARED((NPAD, D_EDGE), jnp.float32),
        ],
        compiler_params=pltpu.CompilerParams(use_tc_tiling_on_sc=False),
    )


BLK = 1000


def _mlp_body(x_ref, ps_ref, pc_ref, w1x_ref, w1a_ref, b1_ref, w2_ref, b2_ref,
              out_ref):
    sums = ps_ref[0] + ps_ref[1]
    cnts = pc_ref[0] + pc_ref[1]
    agg = sums / jnp.maximum(cnts, 1.0)
    h = jnp.dot(x_ref[...], w1x_ref[...], preferred_element_type=jnp.float32)
    h = h + jnp.dot(agg, w1a_ref[...], preferred_element_type=jnp.float32)
    h = jnp.maximum(h + b1_ref[...], 0.0)
    out_ref[...] = (jnp.dot(h, w2_ref[...], preferred_element_type=jnp.float32)
                    + b2_ref[...])


def _mlp_call(x, psums, pcnts, w1xT, w1aT, b1, w2T, b2):
    grid = (N_NODES // BLK,)
    return pl.pallas_call(
        _mlp_body,
        grid=grid,
        in_specs=[
            pl.BlockSpec((BLK, D_FEAT), lambda i: (i, 0)),
            pl.BlockSpec((NC, BLK, D_EDGE), lambda i: (0, i, 0)),
            pl.BlockSpec((NC, BLK, D_EDGE), lambda i: (0, i, 0)),
            pl.BlockSpec((D_FEAT, HIDDEN), lambda i: (0, 0)),
            pl.BlockSpec((D_EDGE, HIDDEN), lambda i: (0, 0)),
            pl.BlockSpec((1, HIDDEN), lambda i: (0, 0)),
            pl.BlockSpec((HIDDEN, D_OUT), lambda i: (0, 0)),
            pl.BlockSpec((1, D_OUT), lambda i: (0, 0)),
        ],
        out_specs=pl.BlockSpec((BLK, D_OUT), lambda i: (i, 0)),
        out_shape=jax.ShapeDtypeStruct((N_NODES, D_OUT), jnp.float32),
    )(x, psums, pcnts, w1xT, w1aT, b1, w2T, b2)


def kernel(x, edge_index, edge_attr, u, batch, W1, b1, W2, b2):
    src = edge_index[0].astype(jnp.int32)
    idx2d = src.reshape(NW * NSUB, SUB)
    psums, pcnts = _make_scatter()(idx2d, edge_attr)
    w1xT = W1[:, :D_FEAT].T
    w1aT = W1[:, D_FEAT:].T
    return _mlp_call(x, psums, pcnts, w1xT, w1aT,
                     b1.reshape(1, HIDDEN), W2.T, b2.reshape(1, D_OUT))
